# baseline (speedup 1.0000x reference)
import math

import jax
import jax.numpy as jnp
from jax.experimental import pallas as pl
from jax.experimental.pallas import tpu as pltpu

HID = 64
B = 64
NL = 2048
NP = 6144
N = NL + NP
R2 = 25.0
TI = 128
TJ = 128
NROW = N // TI

_HIGHEST = jax.lax.Precision.HIGHEST


def _bdot(a, b):
    return jnp.dot(a, b, preferred_element_type=jnp.float32)


def _fused_body(bounds_ref,
                pos8f_ref, posT8f_ref, pos8b_ref, posT8b_ref,
                brow_ref, bcol_ref,
                ligf_ref, protf_ref, t_ref,
                ligW_ref, ligb_ref, protW_ref, protb_ref,
                tmW1_ref, tmb1_ref, tmW2_ref, tmb2_ref,
                relW_ref, relb_ref, rootW_ref,
                roW1_ref, rob1_ref, roW2_ref, rob2_ref,
                out_ref,
                h_ref, hb_ref):
    l = pl.program_id(0)

    @pl.when(l == 0)
    def _prologue():
        half = HID // 2
        e = math.log(10000.0) / (half - 1)
        freqs = jnp.exp(
            jax.lax.broadcasted_iota(jnp.int32, (1, half), 1
                                     ).astype(jnp.float32) * (-e))
        emb = t_ref[:, :] * freqs
        temb = jnp.concatenate([jnp.sin(emb), jnp.cos(emb)], axis=1)
        temb = jax.nn.silu(
            _bdot(temb.astype(jnp.bfloat16), tmW1_ref[:, :]) + tmb1_ref[:, :])
        temb = (_bdot(temb.astype(jnp.bfloat16), tmW2_ref[:, :])
                + tmb2_ref[:, :])
        lb = brow_ref[0:NL, :]
        onehot = (lb == jax.lax.broadcasted_iota(jnp.int32, (1, B), 1)
                  ).astype(jnp.float32)
        t_node = jnp.dot(onehot, temb, precision=_HIGHEST,
                         preferred_element_type=jnp.float32)
        x_lig = (_bdot(ligf_ref[:, :], ligW_ref[:, :])
                 + ligb_ref[:, :]) + t_node
        x_prot = _bdot(protf_ref[:, :], protW_ref[:, :]) + protb_ref[:, :]
        h_ref[0, 0:NL, :] = x_lig
        h_ref[0, NL:N, :] = x_prot

    cur = jax.lax.rem(l, 2)
    nxt = 1 - cur
    relb = relb_ref[pl.ds(l, 1), :]
    relW = relW_ref[l]
    rootW = rootW_ref[l]
    hb_ref[:, :] = h_ref[cur].astype(jnp.bfloat16)

    def row_body(it, carry):
        r0 = it * TI
        p_i = pos8b_ref[pl.ds(r0, TI), :]
        pf_i = pos8f_ref[pl.ds(r0, TI), :]
        p2_i = jnp.sum(pf_i * pf_i, axis=1, keepdims=True)
        b_i = brow_ref[pl.ds(r0, TI), :]

        def col_step(jt):
            j0 = jt * TJ
            pT_j = posT8b_ref[:, pl.ds(j0, TJ)]
            pTf_j = posT8f_ref[:, pl.ds(j0, TJ)]
            p2_j = jnp.sum(pTf_j * pTf_j, axis=0, keepdims=True)
            cross = _bdot(p_i, pT_j)
            d2 = p2_i + p2_j - 2.0 * cross
            b_j = bcol_ref[:, pl.ds(j0, TJ)]
            mask = (d2 < R2) & (b_i == b_j)
            a = mask.astype(jnp.bfloat16)
            return _bdot(a, hb_ref[pl.ds(j0, TJ), :])

        def range_sum(lo, hi, accs):
            half = (hi - lo) // 2

            def body2(k, accs):
                a0, a1 = accs
                jt = lo + 2 * k
                return a0 + col_step(jt), a1 + col_step(jt + 1)

            a0, a1 = jax.lax.fori_loop(0, half, body2, accs)
            a0 = jax.lax.cond(lo + 2 * half < hi,
                              lambda a: a + col_step(hi - 1),
                              lambda a: a, a0)
            return a0, a1

        hb_i = hb_ref[pl.ds(r0, TI), :]
        accs = (-hb_i.astype(jnp.float32), jnp.zeros((TI, HID), jnp.float32))
        accs = range_sum(bounds_ref[0, it], bounds_ref[1, it], accs)
        accs = range_sum(bounds_ref[2, it], bounds_ref[3, it], accs)
        agg = accs[0] + accs[1]
        out_tile = ((_bdot(agg.astype(jnp.bfloat16), relW) + relb)
                    + _bdot(hb_i, rootW))
        h_ref[nxt, pl.ds(r0, TI), :] = out_tile
        return carry

    jax.lax.fori_loop(0, NROW, row_body, 0)

    @pl.when(l == 2)
    def _epilogue():
        xl = h_ref[1, 0:NL, :]
        lb_col = bcol_ref[:, 0:NL]
        onehot = (jax.lax.broadcasted_iota(jnp.int32, (B, 1), 0) == lb_col
                  ).astype(jnp.float32)
        sums = jnp.dot(onehot, xl, precision=_HIGHEST,
                       preferred_element_type=jnp.float32)
        cnt = jnp.sum(onehot, axis=1, keepdims=True)
        mean = sums / jnp.maximum(cnt, 1.0)
        hmid = jax.nn.silu(
            _bdot(mean.astype(jnp.bfloat16), roW1_ref[:, :]) + rob1_ref[:, :])
        out_ref[:, :] = (_bdot(hmid.astype(jnp.bfloat16), roW2_ref[:, :])
                         + rob2_ref[:, :])


@jax.jit
def kernel(lig_pos, lig_feat, prot_pos, prot_feat, t, lig_batch, prot_batch,
           lig_W, lig_b, prot_W, prot_b, tm_W1, tm_b1, tm_W2, tm_b2,
           c1_rel_W, c1_rel_b, c1_root_W, c2_rel_W, c2_rel_b, c2_root_W,
           c3_rel_W, c3_rel_b, c3_root_W, ro_W1, ro_b1, ro_W2, ro_b2):
    bf = jnp.bfloat16
    lig_batch = lig_batch.astype(jnp.int32)
    prot_batch = prot_batch.astype(jnp.int32)
    batch = jnp.concatenate([lig_batch, prot_batch])
    pos = jnp.concatenate([lig_pos, prot_pos], axis=0)
    pos8 = jnp.pad(pos, ((0, 0), (0, 5)))
    posT8 = pos8.T
    brow = batch[:, None]
    bcol = batch[None, :]

    bres = batch.reshape(NROW, TI)
    blo = bres[:, 0]
    bhi = bres[:, -1]
    lj0 = jnp.searchsorted(lig_batch, blo, side='left')
    lj1 = jnp.searchsorted(lig_batch, bhi, side='right')
    pj0 = jnp.searchsorted(prot_batch, blo, side='left')
    pj1 = jnp.searchsorted(prot_batch, bhi, side='right')
    ljt0 = lj0 // TJ
    ljt1 = jnp.where(lj1 > lj0, (lj1 + TJ - 1) // TJ, ljt0)
    nlt = NL // TJ
    pjt0 = nlt + pj0 // TJ
    pjt1 = jnp.where(pj1 > pj0, nlt + (pj1 + TJ - 1) // TJ, pjt0)
    bounds = jnp.stack([ljt0, ljt1, pjt0, pjt1]).astype(jnp.int32)

    relW = jnp.stack([c1_rel_W, c2_rel_W, c3_rel_W]).astype(bf)
    relb = jnp.stack([c1_rel_b, c2_rel_b, c3_rel_b])
    rootW = jnp.stack([c1_root_W, c2_root_W, c3_root_W]).astype(bf)

    smem = pl.BlockSpec(memory_space=pltpu.SMEM)
    out = pl.pallas_call(
        _fused_body,
        grid=(3,),
        in_specs=[smem] + [pl.BlockSpec(memory_space=pltpu.VMEM)] * 24,
        out_specs=pl.BlockSpec(memory_space=pltpu.VMEM),
        out_shape=jax.ShapeDtypeStruct((B, 1), jnp.float32),
        scratch_shapes=[
            pltpu.VMEM((2, N, HID), jnp.float32),
            pltpu.VMEM((N, HID), jnp.bfloat16),
        ],
    )(bounds,
      pos8, posT8, pos8.astype(bf), posT8.astype(bf),
      brow, bcol,
      lig_feat.astype(bf), prot_feat.astype(bf), t[:, None],
      lig_W.astype(bf), lig_b[None, :], prot_W.astype(bf), prot_b[None, :],
      tm_W1.astype(bf), tm_b1[None, :], tm_W2.astype(bf), tm_b2[None, :],
      relW, relb, rootW,
      ro_W1.astype(bf), ro_b1[None, :], ro_W2.astype(bf), ro_b2[None, :])
    return out


# speedup vs baseline: 1.3981x; 1.3981x over previous
import math

import jax
import jax.numpy as jnp
from jax.experimental import pallas as pl
from jax.experimental.pallas import tpu as pltpu

HID = 64
B = 64
NL = 2048
NP = 6144
N = NL + NP
R2 = 25.0
TI = 256
TJ = 256
NROW = N // TI

_HIGHEST = jax.lax.Precision.HIGHEST


def _bdot(a, b):
    return jnp.dot(a, b, preferred_element_type=jnp.float32)


def _fused_body(bounds_ref,
                pos8f_ref, posT8f_ref, pos8b_ref, posT8b_ref,
                brow_ref, bcol_ref,
                ligf_ref, protf_ref, t_ref,
                ligW_ref, ligb_ref, protW_ref, protb_ref,
                tmW1_ref, tmb1_ref, tmW2_ref, tmb2_ref,
                relW_ref, relb_ref, rootW_ref,
                roW1_ref, rob1_ref, roW2_ref, rob2_ref,
                out_ref,
                h_ref, hb_ref, agg_ref):
    l = pl.program_id(0)

    @pl.when(l == 0)
    def _prologue():
        half = HID // 2
        e = math.log(10000.0) / (half - 1)
        freqs = jnp.exp(
            jax.lax.broadcasted_iota(jnp.int32, (1, half), 1
                                     ).astype(jnp.float32) * (-e))
        emb = t_ref[:, :] * freqs
        temb = jnp.concatenate([jnp.sin(emb), jnp.cos(emb)], axis=1)
        temb = jax.nn.silu(
            _bdot(temb.astype(jnp.bfloat16), tmW1_ref[:, :]) + tmb1_ref[:, :])
        temb = (_bdot(temb.astype(jnp.bfloat16), tmW2_ref[:, :])
                + tmb2_ref[:, :])
        lb = brow_ref[0:NL, :]
        onehot = (lb == jax.lax.broadcasted_iota(jnp.int32, (1, B), 1)
                  ).astype(jnp.float32)
        t_node = jnp.dot(onehot, temb, precision=_HIGHEST,
                         preferred_element_type=jnp.float32)
        x_lig = (_bdot(ligf_ref[:, :], ligW_ref[:, :])
                 + ligb_ref[:, :]) + t_node
        x_prot = _bdot(protf_ref[:, :], protW_ref[:, :]) + protb_ref[:, :]
        h_ref[0, 0:NL, :] = x_lig
        h_ref[0, NL:N, :] = x_prot

    cur = jax.lax.rem(l, 2)
    nxt = 1 - cur
    relb = relb_ref[pl.ds(l, 1), :]
    relW = relW_ref[l]
    rootW = rootW_ref[l]
    hb_ref[:, :] = h_ref[cur].astype(jnp.bfloat16)
    agg_ref[:, :] = -hb_ref[:, :].astype(jnp.float32)

    def row_body(it, carry):
        r0 = it * TI
        p_i = pos8b_ref[pl.ds(r0, TI), :]
        pf_i = pos8f_ref[pl.ds(r0, TI), :]
        p2_i = jnp.sum(pf_i * pf_i, axis=1, keepdims=True)
        b_i = brow_ref[pl.ds(r0, TI), :]
        hb_i = hb_ref[pl.ds(r0, TI), :]

        def make_a(jt):
            j0 = jt * TJ
            pT_j = posT8b_ref[:, pl.ds(j0, TJ)]
            pTf_j = posT8f_ref[:, pl.ds(j0, TJ)]
            p2_j = jnp.sum(pTf_j * pTf_j, axis=0, keepdims=True)
            cross = _bdot(p_i, pT_j)
            d2 = p2_i + p2_j - 2.0 * cross
            b_j = bcol_ref[:, pl.ds(j0, TJ)]
            return ((d2 < R2) & (b_i == b_j)).astype(jnp.bfloat16)

        def tri_step(jt, acc):
            j0 = jt * TJ
            a = make_a(jt)
            acc = acc + _bdot(a, hb_ref[pl.ds(j0, TJ), :])
            at = jax.lax.dot_general(
                a, hb_i, (((0,), (0,)), ((), ())),
                preferred_element_type=jnp.float32)
            agg_ref[pl.ds(j0, TJ), :] += at
            return acc

        acc = jnp.zeros((TI, HID), jnp.float32)
        acc = jax.lax.fori_loop(bounds_ref[0, it],
                                jnp.minimum(bounds_ref[1, it], it),
                                tri_step, acc)
        acc = jax.lax.fori_loop(bounds_ref[2, it],
                                jnp.minimum(bounds_ref[3, it], it),
                                tri_step, acc)
        acc = acc + _bdot(make_a(it), hb_i)
        agg_ref[pl.ds(r0, TI), :] += acc
        return carry

    jax.lax.fori_loop(0, NROW, row_body, 0)

    def out_body(it, carry):
        r0 = it * TI
        agg_t = agg_ref[pl.ds(r0, TI), :]
        hb_t = hb_ref[pl.ds(r0, TI), :]
        out_tile = ((_bdot(agg_t.astype(jnp.bfloat16), relW) + relb)
                    + _bdot(hb_t, rootW))
        h_ref[nxt, pl.ds(r0, TI), :] = out_tile
        return carry

    jax.lax.fori_loop(0, NROW, out_body, 0)

    @pl.when(l == 2)
    def _epilogue():
        xl = h_ref[1, 0:NL, :]
        lb_col = bcol_ref[:, 0:NL]
        onehot = (jax.lax.broadcasted_iota(jnp.int32, (B, 1), 0) == lb_col
                  ).astype(jnp.float32)
        sums = jnp.dot(onehot, xl, precision=_HIGHEST,
                       preferred_element_type=jnp.float32)
        cnt = jnp.sum(onehot, axis=1, keepdims=True)
        mean = sums / jnp.maximum(cnt, 1.0)
        hmid = jax.nn.silu(
            _bdot(mean.astype(jnp.bfloat16), roW1_ref[:, :]) + rob1_ref[:, :])
        out_ref[:, :] = (_bdot(hmid.astype(jnp.bfloat16), roW2_ref[:, :])
                         + rob2_ref[:, :])


@jax.jit
def kernel(lig_pos, lig_feat, prot_pos, prot_feat, t, lig_batch, prot_batch,
           lig_W, lig_b, prot_W, prot_b, tm_W1, tm_b1, tm_W2, tm_b2,
           c1_rel_W, c1_rel_b, c1_root_W, c2_rel_W, c2_rel_b, c2_root_W,
           c3_rel_W, c3_rel_b, c3_root_W, ro_W1, ro_b1, ro_W2, ro_b2):
    bf = jnp.bfloat16
    lig_batch = lig_batch.astype(jnp.int32)
    prot_batch = prot_batch.astype(jnp.int32)
    batch = jnp.concatenate([lig_batch, prot_batch])
    pos = jnp.concatenate([lig_pos, prot_pos], axis=0)
    pos8 = jnp.pad(pos, ((0, 0), (0, 5)))
    posT8 = pos8.T
    brow = batch[:, None]
    bcol = batch[None, :]

    bres = batch.reshape(NROW, TI)
    blo = bres[:, 0]
    bhi = bres[:, -1]
    lj0 = jnp.searchsorted(lig_batch, blo, side='left')
    lj1 = jnp.searchsorted(lig_batch, bhi, side='right')
    pj0 = jnp.searchsorted(prot_batch, blo, side='left')
    pj1 = jnp.searchsorted(prot_batch, bhi, side='right')
    ljt0 = lj0 // TJ
    ljt1 = jnp.where(lj1 > lj0, (lj1 + TJ - 1) // TJ, ljt0)
    nlt = NL // TJ
    pjt0 = nlt + pj0 // TJ
    pjt1 = jnp.where(pj1 > pj0, nlt + (pj1 + TJ - 1) // TJ, pjt0)
    bounds = jnp.stack([ljt0, ljt1, pjt0, pjt1]).astype(jnp.int32)

    relW = jnp.stack([c1_rel_W, c2_rel_W, c3_rel_W]).astype(bf)
    relb = jnp.stack([c1_rel_b, c2_rel_b, c3_rel_b])
    rootW = jnp.stack([c1_root_W, c2_root_W, c3_root_W]).astype(bf)

    smem = pl.BlockSpec(memory_space=pltpu.SMEM)
    out = pl.pallas_call(
        _fused_body,
        grid=(3,),
        in_specs=[smem] + [pl.BlockSpec(memory_space=pltpu.VMEM)] * 24,
        out_specs=pl.BlockSpec(memory_space=pltpu.VMEM),
        out_shape=jax.ShapeDtypeStruct((B, 1), jnp.float32),
        scratch_shapes=[
            pltpu.VMEM((2, N, HID), jnp.float32),
            pltpu.VMEM((N, HID), jnp.bfloat16),
            pltpu.VMEM((N, HID), jnp.float32),
        ],
    )(bounds,
      pos8, posT8, pos8.astype(bf), posT8.astype(bf),
      brow, bcol,
      lig_feat.astype(bf), prot_feat.astype(bf), t[:, None],
      lig_W.astype(bf), lig_b[None, :], prot_W.astype(bf), prot_b[None, :],
      tm_W1.astype(bf), tm_b1[None, :], tm_W2.astype(bf), tm_b2[None, :],
      relW, relb, rootW,
      ro_W1.astype(bf), ro_b1[None, :], ro_W2.astype(bf), ro_b2[None, :])
    return out


# speedup vs baseline: 1.4175x; 1.0139x over previous
import math

import jax
import jax.numpy as jnp
from jax.experimental import pallas as pl
from jax.experimental.pallas import tpu as pltpu

HID = 64
B = 64
NL = 2048
NP = 6144
N = NL + NP
R2 = 25.0
TI = 256
TJ = 256
NROW = N // TI

_HIGHEST = jax.lax.Precision.HIGHEST


def _bdot(a, b):
    return jnp.dot(a, b, preferred_element_type=jnp.float32)


def _fused_body(bounds_ref,
                pos8f_ref, posT8f_ref, pos8b_ref, posT8b_ref,
                brow_ref, bcol_ref,
                ligf_ref, protf_ref, t_ref,
                ligW_ref, ligb_ref, protW_ref, protb_ref,
                tmW1_ref, tmb1_ref, tmW2_ref, tmb2_ref,
                relW_ref, relb_ref, rootW_ref,
                roW1_ref, rob1_ref, roW2_ref, rob2_ref,
                out_ref,
                h_ref, hb_ref, agg_ref):
    l = pl.program_id(0)

    @pl.when(l == 0)
    def _prologue():
        half = HID // 2
        e = math.log(10000.0) / (half - 1)
        freqs = jnp.exp(
            jax.lax.broadcasted_iota(jnp.int32, (1, half), 1
                                     ).astype(jnp.float32) * (-e))
        emb = t_ref[:, :] * freqs
        temb = jnp.concatenate([jnp.sin(emb), jnp.cos(emb)], axis=1)
        temb = jax.nn.silu(
            _bdot(temb.astype(jnp.bfloat16), tmW1_ref[:, :]) + tmb1_ref[:, :])
        temb = (_bdot(temb.astype(jnp.bfloat16), tmW2_ref[:, :])
                + tmb2_ref[:, :])
        lb = brow_ref[0:NL, :]
        onehot = (lb == jax.lax.broadcasted_iota(jnp.int32, (1, B), 1)
                  ).astype(jnp.float32)
        t_node = jnp.dot(onehot, temb, precision=_HIGHEST,
                         preferred_element_type=jnp.float32)
        x_lig = (_bdot(ligf_ref[:, :], ligW_ref[:, :])
                 + ligb_ref[:, :]) + t_node
        x_prot = _bdot(protf_ref[:, :], protW_ref[:, :]) + protb_ref[:, :]
        h_ref[0, 0:NL, :] = x_lig
        h_ref[0, NL:N, :] = x_prot

    cur = jax.lax.rem(l, 2)
    nxt = 1 - cur
    relb = relb_ref[pl.ds(l, 1), :]
    relW = relW_ref[l]
    rootW = rootW_ref[l]
    hb_ref[:, :] = h_ref[cur].astype(jnp.bfloat16)
    agg_ref[:, :] = -hb_ref[:, :].astype(jnp.float32)

    def row_body(it, carry):
        r0 = it * TI
        p_i = pos8b_ref[pl.ds(r0, TI), :]
        pf_i = pos8f_ref[pl.ds(r0, TI), :]
        p2_i = jnp.sum(pf_i * pf_i, axis=1, keepdims=True)
        b_i = brow_ref[pl.ds(r0, TI), :]
        hb_i = hb_ref[pl.ds(r0, TI), :]

        def make_a(jt):
            j0 = jt * TJ
            pT_j = posT8b_ref[:, pl.ds(j0, TJ)]
            pTf_j = posT8f_ref[:, pl.ds(j0, TJ)]
            p2_j = jnp.sum(pTf_j * pTf_j, axis=0, keepdims=True)
            cross = _bdot(p_i, pT_j)
            d2 = p2_i + p2_j - 2.0 * cross
            b_j = bcol_ref[:, pl.ds(j0, TJ)]
            return ((d2 < R2) & (b_i == b_j)).astype(jnp.bfloat16)

        def tri_step(jt, acc):
            j0 = jt * TJ
            a = make_a(jt)
            acc = acc + _bdot(a, hb_ref[pl.ds(j0, TJ), :])
            at = jax.lax.dot_general(
                a, hb_i, (((0,), (0,)), ((), ())),
                preferred_element_type=jnp.float32)
            agg_ref[pl.ds(j0, TJ), :] += at
            return acc

        def tri_range(lo, hi, accs):
            hi = jnp.maximum(hi, lo)
            half = (hi - lo) // 2

            def body2(k, accs):
                a0, a1 = accs
                jt = lo + 2 * k
                return tri_step(jt, a0), tri_step(jt + 1, a1)

            a0, a1 = jax.lax.fori_loop(0, half, body2, accs)
            a0 = jax.lax.cond(lo + 2 * half < hi,
                              lambda a: tri_step(hi - 1, a),
                              lambda a: a, a0)
            return a0, a1

        accs = (jnp.zeros((TI, HID), jnp.float32),
                jnp.zeros((TI, HID), jnp.float32))
        accs = tri_range(bounds_ref[0, it],
                         jnp.minimum(bounds_ref[1, it], it), accs)
        accs = tri_range(bounds_ref[2, it],
                         jnp.minimum(bounds_ref[3, it], it), accs)
        acc = (accs[0] + accs[1]) + _bdot(make_a(it), hb_i)
        agg_ref[pl.ds(r0, TI), :] += acc
        return carry

    jax.lax.fori_loop(0, NROW, row_body, 0)

    def out_body(it, carry):
        r0 = it * TI
        agg_t = agg_ref[pl.ds(r0, TI), :]
        hb_t = hb_ref[pl.ds(r0, TI), :]
        out_tile = ((_bdot(agg_t.astype(jnp.bfloat16), relW) + relb)
                    + _bdot(hb_t, rootW))
        h_ref[nxt, pl.ds(r0, TI), :] = out_tile
        return carry

    jax.lax.fori_loop(0, NROW, out_body, 0)

    @pl.when(l == 2)
    def _epilogue():
        xl = h_ref[1, 0:NL, :]
        lb_col = bcol_ref[:, 0:NL]
        onehot = (jax.lax.broadcasted_iota(jnp.int32, (B, 1), 0) == lb_col
                  ).astype(jnp.float32)
        sums = jnp.dot(onehot, xl, precision=_HIGHEST,
                       preferred_element_type=jnp.float32)
        cnt = jnp.sum(onehot, axis=1, keepdims=True)
        mean = sums / jnp.maximum(cnt, 1.0)
        hmid = jax.nn.silu(
            _bdot(mean.astype(jnp.bfloat16), roW1_ref[:, :]) + rob1_ref[:, :])
        out_ref[:, :] = (_bdot(hmid.astype(jnp.bfloat16), roW2_ref[:, :])
                         + rob2_ref[:, :])


@jax.jit
def kernel(lig_pos, lig_feat, prot_pos, prot_feat, t, lig_batch, prot_batch,
           lig_W, lig_b, prot_W, prot_b, tm_W1, tm_b1, tm_W2, tm_b2,
           c1_rel_W, c1_rel_b, c1_root_W, c2_rel_W, c2_rel_b, c2_root_W,
           c3_rel_W, c3_rel_b, c3_root_W, ro_W1, ro_b1, ro_W2, ro_b2):
    bf = jnp.bfloat16
    lig_batch = lig_batch.astype(jnp.int32)
    prot_batch = prot_batch.astype(jnp.int32)
    batch = jnp.concatenate([lig_batch, prot_batch])
    pos = jnp.concatenate([lig_pos, prot_pos], axis=0)
    pos8 = jnp.pad(pos, ((0, 0), (0, 5)))
    posT8 = pos8.T
    brow = batch[:, None]
    bcol = batch[None, :]

    bres = batch.reshape(NROW, TI)
    blo = bres[:, 0]
    bhi = bres[:, -1]
    lj0 = jnp.searchsorted(lig_batch, blo, side='left')
    lj1 = jnp.searchsorted(lig_batch, bhi, side='right')
    pj0 = jnp.searchsorted(prot_batch, blo, side='left')
    pj1 = jnp.searchsorted(prot_batch, bhi, side='right')
    ljt0 = lj0 // TJ
    ljt1 = jnp.where(lj1 > lj0, (lj1 + TJ - 1) // TJ, ljt0)
    nlt = NL // TJ
    pjt0 = nlt + pj0 // TJ
    pjt1 = jnp.where(pj1 > pj0, nlt + (pj1 + TJ - 1) // TJ, pjt0)
    bounds = jnp.stack([ljt0, ljt1, pjt0, pjt1]).astype(jnp.int32)

    relW = jnp.stack([c1_rel_W, c2_rel_W, c3_rel_W]).astype(bf)
    relb = jnp.stack([c1_rel_b, c2_rel_b, c3_rel_b])
    rootW = jnp.stack([c1_root_W, c2_root_W, c3_root_W]).astype(bf)

    smem = pl.BlockSpec(memory_space=pltpu.SMEM)
    out = pl.pallas_call(
        _fused_body,
        grid=(3,),
        in_specs=[smem] + [pl.BlockSpec(memory_space=pltpu.VMEM)] * 24,
        out_specs=pl.BlockSpec(memory_space=pltpu.VMEM),
        out_shape=jax.ShapeDtypeStruct((B, 1), jnp.float32),
        scratch_shapes=[
            pltpu.VMEM((2, N, HID), jnp.float32),
            pltpu.VMEM((N, HID), jnp.bfloat16),
            pltpu.VMEM((N, HID), jnp.float32),
        ],
    )(bounds,
      pos8, posT8, pos8.astype(bf), posT8.astype(bf),
      brow, bcol,
      lig_feat.astype(bf), prot_feat.astype(bf), t[:, None],
      lig_W.astype(bf), lig_b[None, :], prot_W.astype(bf), prot_b[None, :],
      tm_W1.astype(bf), tm_b1[None, :], tm_W2.astype(bf), tm_b2[None, :],
      relW, relb, rootW,
      ro_W1.astype(bf), ro_b1[None, :], ro_W2.astype(bf), ro_b2[None, :])
    return out


# speedup vs baseline: 1.5162x; 1.0696x over previous
import math

import jax
import jax.numpy as jnp
from jax.experimental import pallas as pl
from jax.experimental.pallas import tpu as pltpu

HID = 64
B = 64
NL = 2048
NP = 6144
N = NL + NP
R2 = 25.0
TI = 256
TJ = 256
NROW = N // TI
ACK = 128

_HIGHEST = jax.lax.Precision.HIGHEST


def _bdot(a, b):
    return jnp.dot(a, b, preferred_element_type=jnp.float32)


def _fused_body(bounds_ref,
                pos8f_ref, posT8f_ref, pos8b_ref, posT8b_ref,
                brow_ref, bcol_ref,
                ligf_ref, protf_ref, t_ref,
                ligW_ref, ligb_ref, protW_ref, protb_ref,
                tmW1_ref, tmb1_ref, tmW2_ref, tmb2_ref,
                relW_ref, relb_ref, rootW_ref,
                roW1_ref, rob1_ref, roW2_ref, rob2_ref,
                out_ref,
                h_ref, hb_ref, agg_ref, acache_ref, idx_ref):
    l = pl.program_id(0)

    @pl.when(l == 0)
    def _prologue():
        half = HID // 2
        e = math.log(10000.0) / (half - 1)
        freqs = jnp.exp(
            jax.lax.broadcasted_iota(jnp.int32, (1, half), 1
                                     ).astype(jnp.float32) * (-e))
        emb = t_ref[:, :] * freqs
        temb = jnp.concatenate([jnp.sin(emb), jnp.cos(emb)], axis=1)
        temb = jax.nn.silu(
            _bdot(temb.astype(jnp.bfloat16), tmW1_ref[:, :]) + tmb1_ref[:, :])
        temb = (_bdot(temb.astype(jnp.bfloat16), tmW2_ref[:, :])
                + tmb2_ref[:, :])
        lb = brow_ref[0:NL, :]
        onehot = (lb == jax.lax.broadcasted_iota(jnp.int32, (1, B), 1)
                  ).astype(jnp.float32)
        t_node = jnp.dot(onehot, temb, precision=_HIGHEST,
                         preferred_element_type=jnp.float32)
        x_lig = (_bdot(ligf_ref[:, :], ligW_ref[:, :])
                 + ligb_ref[:, :]) + t_node
        x_prot = _bdot(protf_ref[:, :], protW_ref[:, :]) + protb_ref[:, :]
        h_ref[0, 0:NL, :] = x_lig
        h_ref[0, NL:N, :] = x_prot

    cur = jax.lax.rem(l, 2)
    nxt = 1 - cur
    relb = relb_ref[pl.ds(l, 1), :]
    relW = relW_ref[l]
    rootW = rootW_ref[l]
    hb_ref[:, :] = h_ref[cur].astype(jnp.bfloat16)
    agg_ref[:, :] = -hb_ref[:, :].astype(jnp.float32)
    idx_ref[0] = 0

    def row_body(it, carry):
        r0 = it * TI
        p_i = pos8b_ref[pl.ds(r0, TI), :]
        pf_i = pos8f_ref[pl.ds(r0, TI), :]
        p2_i = jnp.sum(pf_i * pf_i, axis=1, keepdims=True)
        b_i = brow_ref[pl.ds(r0, TI), :]
        hb_i = hb_ref[pl.ds(r0, TI), :]

        def make_a(jt):
            j0 = jt * TJ
            pT_j = posT8b_ref[:, pl.ds(j0, TJ)]
            pTf_j = posT8f_ref[:, pl.ds(j0, TJ)]
            p2_j = jnp.sum(pTf_j * pTf_j, axis=0, keepdims=True)
            cross = _bdot(p_i, pT_j)
            d2 = p2_i + p2_j - 2.0 * cross
            b_j = bcol_ref[:, pl.ds(j0, TJ)]
            return ((d2 < R2) & (b_i == b_j)).astype(jnp.bfloat16)

        def get_a(jt):
            idx = idx_ref[0]
            idx_ref[0] = idx + 1
            ic = jnp.minimum(idx, ACK - 1)
            a = jax.lax.cond(jnp.logical_and(l > 0, idx < ACK),
                             lambda: acache_ref[ic],
                             lambda: make_a(jt))

            @pl.when(jnp.logical_and(l == 0, idx < ACK))
            def _store():
                acache_ref[ic] = a

            return a

        def tri_step(jt, acc):
            j0 = jt * TJ
            a = get_a(jt)
            acc = acc + _bdot(a, hb_ref[pl.ds(j0, TJ), :])
            at = jax.lax.dot_general(
                a, hb_i, (((0,), (0,)), ((), ())),
                preferred_element_type=jnp.float32)
            agg_ref[pl.ds(j0, TJ), :] += at
            return acc

        def tri_range(lo, hi, accs):
            hi = jnp.maximum(hi, lo)
            half = (hi - lo) // 2

            def body2(k, accs):
                a0, a1 = accs
                jt = lo + 2 * k
                return tri_step(jt, a0), tri_step(jt + 1, a1)

            a0, a1 = jax.lax.fori_loop(0, half, body2, accs)
            a0 = jax.lax.cond(lo + 2 * half < hi,
                              lambda a: tri_step(hi - 1, a),
                              lambda a: a, a0)
            return a0, a1

        accs = (jnp.zeros((TI, HID), jnp.float32),
                jnp.zeros((TI, HID), jnp.float32))
        accs = tri_range(bounds_ref[0, it],
                         jnp.minimum(bounds_ref[1, it], it), accs)
        accs = tri_range(bounds_ref[2, it],
                         jnp.minimum(bounds_ref[3, it], it), accs)
        acc = (accs[0] + accs[1]) + _bdot(get_a(it), hb_i)
        agg_ref[pl.ds(r0, TI), :] += acc
        return carry

    jax.lax.fori_loop(0, NROW, row_body, 0)

    def out_body(it, carry):
        r0 = it * TI
        agg_t = agg_ref[pl.ds(r0, TI), :]
        hb_t = hb_ref[pl.ds(r0, TI), :]
        out_tile = ((_bdot(agg_t.astype(jnp.bfloat16), relW) + relb)
                    + _bdot(hb_t, rootW))
        h_ref[nxt, pl.ds(r0, TI), :] = out_tile
        return carry

    jax.lax.fori_loop(0, NROW, out_body, 0)

    @pl.when(l == 2)
    def _epilogue():
        xl = h_ref[1, 0:NL, :]
        lb_col = bcol_ref[:, 0:NL]
        onehot = (jax.lax.broadcasted_iota(jnp.int32, (B, 1), 0) == lb_col
                  ).astype(jnp.float32)
        sums = jnp.dot(onehot, xl, precision=_HIGHEST,
                       preferred_element_type=jnp.float32)
        cnt = jnp.sum(onehot, axis=1, keepdims=True)
        mean = sums / jnp.maximum(cnt, 1.0)
        hmid = jax.nn.silu(
            _bdot(mean.astype(jnp.bfloat16), roW1_ref[:, :]) + rob1_ref[:, :])
        out_ref[:, :] = (_bdot(hmid.astype(jnp.bfloat16), roW2_ref[:, :])
                         + rob2_ref[:, :])


@jax.jit
def kernel(lig_pos, lig_feat, prot_pos, prot_feat, t, lig_batch, prot_batch,
           lig_W, lig_b, prot_W, prot_b, tm_W1, tm_b1, tm_W2, tm_b2,
           c1_rel_W, c1_rel_b, c1_root_W, c2_rel_W, c2_rel_b, c2_root_W,
           c3_rel_W, c3_rel_b, c3_root_W, ro_W1, ro_b1, ro_W2, ro_b2):
    bf = jnp.bfloat16
    lig_batch = lig_batch.astype(jnp.int32)
    prot_batch = prot_batch.astype(jnp.int32)
    batch = jnp.concatenate([lig_batch, prot_batch])
    pos = jnp.concatenate([lig_pos, prot_pos], axis=0)
    pos8 = jnp.pad(pos, ((0, 0), (0, 5)))
    posT8 = pos8.T
    brow = batch[:, None]
    bcol = batch[None, :]

    bres = batch.reshape(NROW, TI)
    blo = bres[:, 0]
    bhi = bres[:, -1]
    lj0 = jnp.searchsorted(lig_batch, blo, side='left')
    lj1 = jnp.searchsorted(lig_batch, bhi, side='right')
    pj0 = jnp.searchsorted(prot_batch, blo, side='left')
    pj1 = jnp.searchsorted(prot_batch, bhi, side='right')
    ljt0 = lj0 // TJ
    ljt1 = jnp.where(lj1 > lj0, (lj1 + TJ - 1) // TJ, ljt0)
    nlt = NL // TJ
    pjt0 = nlt + pj0 // TJ
    pjt1 = jnp.where(pj1 > pj0, nlt + (pj1 + TJ - 1) // TJ, pjt0)
    bounds = jnp.stack([ljt0, ljt1, pjt0, pjt1]).astype(jnp.int32)

    relW = jnp.stack([c1_rel_W, c2_rel_W, c3_rel_W]).astype(bf)
    relb = jnp.stack([c1_rel_b, c2_rel_b, c3_rel_b])
    rootW = jnp.stack([c1_root_W, c2_root_W, c3_root_W]).astype(bf)

    smem = pl.BlockSpec(memory_space=pltpu.SMEM)
    out = pl.pallas_call(
        _fused_body,
        grid=(3,),
        in_specs=[smem] + [pl.BlockSpec(memory_space=pltpu.VMEM)] * 24,
        out_specs=pl.BlockSpec(memory_space=pltpu.VMEM),
        out_shape=jax.ShapeDtypeStruct((B, 1), jnp.float32),
        scratch_shapes=[
            pltpu.VMEM((2, N, HID), jnp.float32),
            pltpu.VMEM((N, HID), jnp.bfloat16),
            pltpu.VMEM((N, HID), jnp.float32),
            pltpu.VMEM((ACK, TI, TJ), jnp.bfloat16),
            pltpu.SMEM((1,), jnp.int32),
        ],
    )(bounds,
      pos8, posT8, pos8.astype(bf), posT8.astype(bf),
      brow, bcol,
      lig_feat.astype(bf), prot_feat.astype(bf), t[:, None],
      lig_W.astype(bf), lig_b[None, :], prot_W.astype(bf), prot_b[None, :],
      tm_W1.astype(bf), tm_b1[None, :], tm_W2.astype(bf), tm_b2[None, :],
      relW, relb, rootW,
      ro_W1.astype(bf), ro_b1[None, :], ro_W2.astype(bf), ro_b2[None, :])
    return out


# speedup vs baseline: 1.5234x; 1.0047x over previous
import math

import jax
import jax.numpy as jnp
from jax.experimental import pallas as pl
from jax.experimental.pallas import tpu as pltpu

HID = 64
B = 64
NL = 2048
NP = 6144
N = NL + NP
R2 = 25.0
TI = 256
TJ = 256
NROW = N // TI
ACK = 128

_HIGHEST = jax.lax.Precision.HIGHEST


def _bdot(a, b):
    return jnp.dot(a, b, preferred_element_type=jnp.float32)


def _fused_body(bounds_ref,
                pos8f_ref, posT8f_ref, pos8b_ref, posT8b_ref,
                brow_ref, bcol_ref,
                ligf_ref, protf_ref, t_ref,
                ligW_ref, ligb_ref, protW_ref, protb_ref,
                tmW1_ref, tmb1_ref, tmW2_ref, tmb2_ref,
                relW_ref, relb_ref, rootW_ref,
                roW1_ref, rob1_ref, roW2_ref, rob2_ref,
                out_ref,
                h_ref, hb_ref, agg_ref, aggT_ref, hbT_ref, acache_ref,
                idx_ref):
    l = pl.program_id(0)

    @pl.when(l == 0)
    def _prologue():
        half = HID // 2
        e = math.log(10000.0) / (half - 1)
        freqs = jnp.exp(
            jax.lax.broadcasted_iota(jnp.int32, (1, half), 1
                                     ).astype(jnp.float32) * (-e))
        emb = t_ref[:, :] * freqs
        temb = jnp.concatenate([jnp.sin(emb), jnp.cos(emb)], axis=1)
        temb = jax.nn.silu(
            _bdot(temb.astype(jnp.bfloat16), tmW1_ref[:, :]) + tmb1_ref[:, :])
        temb = (_bdot(temb.astype(jnp.bfloat16), tmW2_ref[:, :])
                + tmb2_ref[:, :])
        lb = brow_ref[0:NL, :]
        onehot = (lb == jax.lax.broadcasted_iota(jnp.int32, (1, B), 1)
                  ).astype(jnp.float32)
        t_node = jnp.dot(onehot, temb, precision=_HIGHEST,
                         preferred_element_type=jnp.float32)
        x_lig = (_bdot(ligf_ref[:, :], ligW_ref[:, :])
                 + ligb_ref[:, :]) + t_node
        x_prot = _bdot(protf_ref[:, :], protW_ref[:, :]) + protb_ref[:, :]
        h_ref[0, 0:NL, :] = x_lig
        h_ref[0, NL:N, :] = x_prot

    cur = jax.lax.rem(l, 2)
    nxt = 1 - cur
    relb = relb_ref[pl.ds(l, 1), :]
    relW = relW_ref[l]
    rootW = rootW_ref[l]
    hb_ref[:, :] = h_ref[cur].astype(jnp.bfloat16)
    hbT_ref[:, :] = jnp.swapaxes(hb_ref[:, :], 0, 1)
    agg_ref[:, :] = -hb_ref[:, :].astype(jnp.float32)
    aggT_ref[:, :] = jnp.zeros((HID, N), jnp.float32)
    idx_ref[0] = 0

    def row_body(it, carry):
        r0 = it * TI
        p_i = pos8b_ref[pl.ds(r0, TI), :]
        pf_i = pos8f_ref[pl.ds(r0, TI), :]
        p2_i = jnp.sum(pf_i * pf_i, axis=1, keepdims=True)
        b_i = brow_ref[pl.ds(r0, TI), :]
        hb_i = hb_ref[pl.ds(r0, TI), :]

        def make_a(jt):
            j0 = jt * TJ
            pT_j = posT8b_ref[:, pl.ds(j0, TJ)]
            pTf_j = posT8f_ref[:, pl.ds(j0, TJ)]
            p2_j = jnp.sum(pTf_j * pTf_j, axis=0, keepdims=True)
            cross = _bdot(p_i, pT_j)
            d2 = p2_i + p2_j - 2.0 * cross
            b_j = bcol_ref[:, pl.ds(j0, TJ)]
            return ((d2 < R2) & (b_i == b_j)).astype(jnp.bfloat16)

        def get_a(jt):
            idx = idx_ref[0]
            idx_ref[0] = idx + 1
            ic = jnp.minimum(idx, ACK - 1)
            a = jax.lax.cond(jnp.logical_and(l > 0, idx < ACK),
                             lambda: acache_ref[ic],
                             lambda: make_a(jt))

            @pl.when(jnp.logical_and(l == 0, idx < ACK))
            def _store():
                acache_ref[ic] = a

            return a

        hbT_i = hbT_ref[:, pl.ds(r0, TI)]

        def tri_step(jt, acc):
            j0 = jt * TJ
            a = get_a(jt)
            acc = acc + _bdot(a, hb_ref[pl.ds(j0, TJ), :])
            aggT_ref[:, pl.ds(j0, TJ)] += _bdot(hbT_i, a)
            return acc

        def tri_range(lo, hi, accs):
            hi = jnp.maximum(hi, lo)
            half = (hi - lo) // 2

            def body2(k, accs):
                a0, a1 = accs
                jt = lo + 2 * k
                return tri_step(jt, a0), tri_step(jt + 1, a1)

            a0, a1 = jax.lax.fori_loop(0, half, body2, accs)
            a0 = jax.lax.cond(lo + 2 * half < hi,
                              lambda a: tri_step(hi - 1, a),
                              lambda a: a, a0)
            return a0, a1

        accs = (jnp.zeros((TI, HID), jnp.float32),
                jnp.zeros((TI, HID), jnp.float32))
        accs = tri_range(bounds_ref[0, it],
                         jnp.minimum(bounds_ref[1, it], it), accs)
        accs = tri_range(bounds_ref[2, it],
                         jnp.minimum(bounds_ref[3, it], it), accs)
        acc = (accs[0] + accs[1]) + _bdot(get_a(it), hb_i)
        agg_ref[pl.ds(r0, TI), :] += acc
        return carry

    jax.lax.fori_loop(0, NROW, row_body, 0)

    def out_body(it, carry):
        r0 = it * TI
        agg_t = (agg_ref[pl.ds(r0, TI), :]
                 + jnp.swapaxes(aggT_ref[:, pl.ds(r0, TI)], 0, 1))
        hb_t = hb_ref[pl.ds(r0, TI), :]
        out_tile = ((_bdot(agg_t.astype(jnp.bfloat16), relW) + relb)
                    + _bdot(hb_t, rootW))
        h_ref[nxt, pl.ds(r0, TI), :] = out_tile
        return carry

    jax.lax.fori_loop(0, NROW, out_body, 0)

    @pl.when(l == 2)
    def _epilogue():
        xl = h_ref[1, 0:NL, :]
        lb_col = bcol_ref[:, 0:NL]
        onehot = (jax.lax.broadcasted_iota(jnp.int32, (B, 1), 0) == lb_col
                  ).astype(jnp.float32)
        sums = jnp.dot(onehot, xl, precision=_HIGHEST,
                       preferred_element_type=jnp.float32)
        cnt = jnp.sum(onehot, axis=1, keepdims=True)
        mean = sums / jnp.maximum(cnt, 1.0)
        hmid = jax.nn.silu(
            _bdot(mean.astype(jnp.bfloat16), roW1_ref[:, :]) + rob1_ref[:, :])
        out_ref[:, :] = (_bdot(hmid.astype(jnp.bfloat16), roW2_ref[:, :])
                         + rob2_ref[:, :])


@jax.jit
def kernel(lig_pos, lig_feat, prot_pos, prot_feat, t, lig_batch, prot_batch,
           lig_W, lig_b, prot_W, prot_b, tm_W1, tm_b1, tm_W2, tm_b2,
           c1_rel_W, c1_rel_b, c1_root_W, c2_rel_W, c2_rel_b, c2_root_W,
           c3_rel_W, c3_rel_b, c3_root_W, ro_W1, ro_b1, ro_W2, ro_b2):
    bf = jnp.bfloat16
    lig_batch = lig_batch.astype(jnp.int32)
    prot_batch = prot_batch.astype(jnp.int32)
    batch = jnp.concatenate([lig_batch, prot_batch])
    pos = jnp.concatenate([lig_pos, prot_pos], axis=0)
    pos8 = jnp.pad(pos, ((0, 0), (0, 5)))
    posT8 = pos8.T
    brow = batch[:, None]
    bcol = batch[None, :]

    bres = batch.reshape(NROW, TI)
    blo = bres[:, 0]
    bhi = bres[:, -1]
    lj0 = jnp.searchsorted(lig_batch, blo, side='left')
    lj1 = jnp.searchsorted(lig_batch, bhi, side='right')
    pj0 = jnp.searchsorted(prot_batch, blo, side='left')
    pj1 = jnp.searchsorted(prot_batch, bhi, side='right')
    ljt0 = lj0 // TJ
    ljt1 = jnp.where(lj1 > lj0, (lj1 + TJ - 1) // TJ, ljt0)
    nlt = NL // TJ
    pjt0 = nlt + pj0 // TJ
    pjt1 = jnp.where(pj1 > pj0, nlt + (pj1 + TJ - 1) // TJ, pjt0)
    bounds = jnp.stack([ljt0, ljt1, pjt0, pjt1]).astype(jnp.int32)

    relW = jnp.stack([c1_rel_W, c2_rel_W, c3_rel_W]).astype(bf)
    relb = jnp.stack([c1_rel_b, c2_rel_b, c3_rel_b])
    rootW = jnp.stack([c1_root_W, c2_root_W, c3_root_W]).astype(bf)

    smem = pl.BlockSpec(memory_space=pltpu.SMEM)
    out = pl.pallas_call(
        _fused_body,
        grid=(3,),
        in_specs=[smem] + [pl.BlockSpec(memory_space=pltpu.VMEM)] * 24,
        out_specs=pl.BlockSpec(memory_space=pltpu.VMEM),
        out_shape=jax.ShapeDtypeStruct((B, 1), jnp.float32),
        scratch_shapes=[
            pltpu.VMEM((2, N, HID), jnp.float32),
            pltpu.VMEM((N, HID), jnp.bfloat16),
            pltpu.VMEM((N, HID), jnp.float32),
            pltpu.VMEM((HID, N), jnp.float32),
            pltpu.VMEM((HID, N), jnp.bfloat16),
            pltpu.VMEM((ACK, TI, TJ), jnp.bfloat16),
            pltpu.SMEM((1,), jnp.int32),
        ],
    )(bounds,
      pos8, posT8, pos8.astype(bf), posT8.astype(bf),
      brow, bcol,
      lig_feat.astype(bf), prot_feat.astype(bf), t[:, None],
      lig_W.astype(bf), lig_b[None, :], prot_W.astype(bf), prot_b[None, :],
      tm_W1.astype(bf), tm_b1[None, :], tm_W2.astype(bf), tm_b2[None, :],
      relW, relb, rootW,
      ro_W1.astype(bf), ro_b1[None, :], ro_W2.astype(bf), ro_b2[None, :])
    return out
